# baseline (speedup 1.0000x reference)
import functools

import jax
import jax.numpy as jnp
from jax import lax
from jax.experimental import pallas as pl
from jax.experimental.pallas import tpu as pltpu
from jax.experimental.pallas import tpu_sc as plsc

N = 10000
E = 320000
D = 128
NC = 2
NS = 16
NW = NC * NS
BATCH = 128
NB = 79
EPT = NB * BATCH
NB0 = 109
NB1 = 49
E_PAD = NW * EPT
N_PAD = 10240
RPT = N_PAD // NS

_mesh = plsc.VectorSubcoreMesh(core_axis_name="c", subcore_axis_name="s")


def _hist_body(dst_hbm, deg_out, idx_v, hist_v):
    c = lax.axis_index("c")
    s = lax.axis_index("s")
    w = c * NS + s
    pltpu.sync_copy(dst_hbm.at[w], idx_v)
    zero16 = jnp.zeros((16,), jnp.float32)

    @pl.loop(0, N_PAD // 16, unroll=8)
    def _zero(i):
        hist_v[0, pl.ds(i * 16, 16)] = zero16

    ones = jnp.ones((16,), jnp.float32)
    zidx = jnp.zeros((16,), jnp.int32)

    @pl.loop(0, NB)
    def _batch(b):
        @pl.loop(0, BATCH // 16)
        def _chunk(j):
            idx = idx_v[b, pl.ds(j * 16, 16)]
            plsc.addupdate_scatter(hist_v, [zidx, idx], ones)

    pltpu.sync_copy(hist_v, deg_out.at[w])


_sc_params = pltpu.CompilerParams(needs_layout_passes=False)

_hist = pl.kernel(
    _hist_body,
    out_type=jax.ShapeDtypeStruct((NW, 1, N_PAD), jnp.float32),
    mesh=_mesh,
    compiler_params=_sc_params,
    scratch_types=[
        pltpu.VMEM((NB, BATCH), jnp.int32),
        pltpu.VMEM((1, N_PAD), jnp.float32),
    ],
)


def _prop_body(y_hbm, srcA_hbm, dstA_hbm, srcB_hbm, dstB_hbm, parts_out,
               src_v, dst_v, rows_v, accum, sem):
    c = lax.axis_index("c")
    s = lax.axis_index("s")

    zero16 = jnp.zeros((16,), jnp.float32)

    @pl.loop(0, BATCH)
    def _zrow(r):
        @pl.loop(0, D // 16, unroll=8)
        def _zcol(j):
            rows_v[r, pl.ds(j * 16, 16)] = zero16

    @pl.loop(0, RPT // BATCH)
    def _zchunk(k):
        pltpu.sync_copy(
            rows_v, accum.at[pl.ds(s * RPT + k * BATCH, BATCH), :])

    plsc.subcore_barrier()

    @pl.when(c == 0)
    def _sc0():
        pltpu.sync_copy(srcA_hbm.at[s], src_v)
        pltpu.sync_copy(dstA_hbm.at[s], dst_v)

        @pl.loop(0, NB0)
        def _batch(b):
            pltpu.async_copy(y_hbm.at[src_v.at[b]], rows_v, sem).wait()
            pltpu.sync_copy(rows_v, accum.at[dst_v.at[b]], add=True)

    @pl.when(c == 1)
    def _sc1():
        pltpu.sync_copy(srcB_hbm.at[s], src_v.at[pl.ds(0, NB1), :])
        pltpu.sync_copy(dstB_hbm.at[s], dst_v.at[pl.ds(0, NB1), :])

        @pl.loop(0, NB1)
        def _batch(b):
            pltpu.async_copy(y_hbm.at[src_v.at[b]], rows_v, sem).wait()
            pltpu.sync_copy(rows_v, accum.at[dst_v.at[b]], add=True)

    plsc.subcore_barrier()

    @pl.loop(0, RPT // BATCH)
    def _read(k):
        pltpu.sync_copy(
            accum.at[pl.ds(s * RPT + k * BATCH, BATCH), :], rows_v)
        pltpu.sync_copy(
            rows_v, parts_out.at[c, pl.ds(s * RPT + k * BATCH, BATCH), :])


_prop = pl.kernel(
    _prop_body,
    out_type=jax.ShapeDtypeStruct((NC, N_PAD, D), jnp.float32),
    mesh=_mesh,
    compiler_params=_sc_params,
    scratch_types=[
        pltpu.VMEM((NB0, BATCH), jnp.int32),
        pltpu.VMEM((NB0, BATCH), jnp.int32),
        pltpu.VMEM((BATCH, D), jnp.float32),
        pltpu.VMEM_SHARED((N_PAD, D), jnp.float32),
        pltpu.SemaphoreType.DMA,
    ],
)


BR = 1000


def _scale_body(degT_ref, x_ref, y_ref):
    deg = jnp.sum(degT_ref[...], axis=1, keepdims=True) + 1.0
    y_ref[...] = x_ref[...] * lax.rsqrt(deg)


def _scale(degT, x):
    grid = N // BR
    return pl.pallas_call(
        _scale_body,
        grid=(grid,),
        in_specs=[
            pl.BlockSpec((BR, NW), lambda i: (i, 0)),
            pl.BlockSpec((BR, D), lambda i: (i, 0)),
        ],
        out_specs=pl.BlockSpec((BR, D), lambda i: (i, 0)),
        out_shape=jax.ShapeDtypeStruct((N, D), jnp.float32),
    )(degT, x)


def _cell_body(degT_ref, sa_ref, sb_ref, v_ref, wc0_ref, bc0_ref, wl0_ref,
               bl0_ref, wc2_ref, bc2_ref, wl2_ref, bl2_ref, *out_refs,
               relu_out, emit_y):
    deg = jnp.sum(degT_ref[...], axis=1, keepdims=True) + 1.0
    dinv = lax.rsqrt(deg)
    v = v_ref[...]
    px = dinv * (sa_ref[...] + sb_ref[...]) + v / deg
    t = jnp.dot(px, wc0_ref[...], preferred_element_type=jnp.float32)
    t = t + bc0_ref[...]
    z = jax.nn.sigmoid(
        jnp.dot(t, wl0_ref[...], preferred_element_type=jnp.float32)
        + bl0_ref[...])
    u = jnp.dot(px, wc2_ref[...], preferred_element_type=jnp.float32)
    u = u + bc2_ref[...]
    ht = jnp.tanh(
        jnp.dot(u, wl2_ref[...], preferred_element_type=jnp.float32)
        + bl2_ref[...])
    h = (1.0 - z) * ht
    if relu_out:
        h = jnp.maximum(h, 0.0)
    out_refs[0][...] = h
    if emit_y:
        out_refs[1][...] = h * dinv


def _cell(degT, sa, sb, v, Wc, bc, Wl, bl, relu_out, emit_y):
    grid = N // BR
    row = lambda i: (i, 0)
    full = lambda i: (0, 0)
    out_shape = [jax.ShapeDtypeStruct((N, D), jnp.float32)]
    out_specs = [pl.BlockSpec((BR, D), row)]
    if emit_y:
        out_shape.append(jax.ShapeDtypeStruct((N, D), jnp.float32))
        out_specs.append(pl.BlockSpec((BR, D), row))
    body = functools.partial(_cell_body, relu_out=relu_out, emit_y=emit_y)
    return pl.pallas_call(
        body,
        grid=(grid,),
        in_specs=[
            pl.BlockSpec((BR, NW), row),
            pl.BlockSpec((BR, D), row),
            pl.BlockSpec((BR, D), row),
            pl.BlockSpec((BR, D), row),
            pl.BlockSpec((D, D), full),
            pl.BlockSpec((1, D), full),
            pl.BlockSpec((D, D), full),
            pl.BlockSpec((1, D), full),
            pl.BlockSpec((D, D), full),
            pl.BlockSpec((1, D), full),
            pl.BlockSpec((D, D), full),
            pl.BlockSpec((1, D), full),
        ],
        out_specs=out_specs,
        out_shape=out_shape,
    )(degT, sa, sb, v,
      Wc[0], bc[0].reshape(1, D), Wl[0][:D], bl[0].reshape(1, D),
      Wc[2], bc[2].reshape(1, D), Wl[2][:D], bl[2].reshape(1, D))


def kernel(x, edge_index, Wc1, bc1, Wl1, bl1, Wc2, bc2, Wl2, bl2):
    src = edge_index[0]
    dst = edge_index[1]
    pad = E_PAD - E
    src3 = jnp.concatenate(
        [src, jnp.zeros((pad,), jnp.int32)]).reshape(NW, NB, BATCH)
    dst3 = jnp.concatenate(
        [dst, jnp.full((pad,), N, jnp.int32)]).reshape(NW, NB, BATCH)
    na = NS * NB0 * BATCH
    ntot = na + NS * NB1 * BATCH
    padg = ntot - E
    srcp = jnp.concatenate([src, jnp.zeros((padg,), jnp.int32)])
    dstp = jnp.concatenate([dst, jnp.full((padg,), N, jnp.int32)])
    srcA = srcp[:na].reshape(NS, NB0, BATCH)
    dstA = dstp[:na].reshape(NS, NB0, BATCH)
    srcB = srcp[na:].reshape(NS, NB1, BATCH)
    dstB = dstp[na:].reshape(NS, NB1, BATCH)

    deg_parts = _hist(dst3).reshape(NW, N_PAD)
    degT = jnp.transpose(deg_parts)[:N]

    y1 = _scale(degT, x)
    parts1 = _prop(y1, srcA, dstA, srcB, dstB)
    h1, y2 = _cell(degT, parts1[0, :N], parts1[1, :N], x, Wc1, bc1, Wl1, bl1,
                   relu_out=True, emit_y=True)
    parts2 = _prop(y2, srcA, dstA, srcB, dstB)
    (h2,) = _cell(degT, parts2[0, :N], parts2[1, :N], h1, Wc2, bc2, Wl2, bl2,
                  relu_out=False, emit_y=False)
    return h2


# speedup vs baseline: 1.0033x; 1.0033x over previous
import functools

import jax
import jax.numpy as jnp
from jax import lax
from jax.experimental import pallas as pl
from jax.experimental.pallas import tpu as pltpu
from jax.experimental.pallas import tpu_sc as plsc

N = 10000
E = 320000
D = 128
NC = 2
NS = 16
NW = NC * NS
BATCH = 128
NB = 79
EPT = NB * BATCH
NB0 = 103
NB1 = 55
E_PAD = NW * EPT
N_PAD = 10240
RPT = N_PAD // NS

_mesh = plsc.VectorSubcoreMesh(core_axis_name="c", subcore_axis_name="s")


def _hist_body(dst_hbm, deg_out, idx_v, hist_v):
    c = lax.axis_index("c")
    s = lax.axis_index("s")
    w = c * NS + s
    pltpu.sync_copy(dst_hbm.at[w], idx_v)
    zero16 = jnp.zeros((16,), jnp.float32)

    @pl.loop(0, N_PAD // 16, unroll=8)
    def _zero(i):
        hist_v[0, pl.ds(i * 16, 16)] = zero16

    ones = jnp.ones((16,), jnp.float32)
    zidx = jnp.zeros((16,), jnp.int32)

    @pl.loop(0, NB)
    def _batch(b):
        @pl.loop(0, BATCH // 16)
        def _chunk(j):
            idx = idx_v[b, pl.ds(j * 16, 16)]
            plsc.addupdate_scatter(hist_v, [zidx, idx], ones)

    pltpu.sync_copy(hist_v, deg_out.at[w])


_sc_params = pltpu.CompilerParams(needs_layout_passes=False)

_hist = pl.kernel(
    _hist_body,
    out_type=jax.ShapeDtypeStruct((NW, 1, N_PAD), jnp.float32),
    mesh=_mesh,
    compiler_params=_sc_params,
    scratch_types=[
        pltpu.VMEM((NB, BATCH), jnp.int32),
        pltpu.VMEM((1, N_PAD), jnp.float32),
    ],
)


def _prop_body(y_hbm, srcA_hbm, dstA_hbm, srcB_hbm, dstB_hbm, parts_out,
               src_v, dst_v, rows_v, accum, sem):
    c = lax.axis_index("c")
    s = lax.axis_index("s")

    zero16 = jnp.zeros((16,), jnp.float32)

    @pl.loop(0, BATCH)
    def _zrow(r):
        @pl.loop(0, D // 16, unroll=8)
        def _zcol(j):
            rows_v[r, pl.ds(j * 16, 16)] = zero16

    @pl.loop(0, RPT // BATCH)
    def _zchunk(k):
        pltpu.sync_copy(
            rows_v, accum.at[pl.ds(s * RPT + k * BATCH, BATCH), :])

    plsc.subcore_barrier()

    @pl.when(c == 0)
    def _sc0():
        pltpu.sync_copy(srcA_hbm.at[s], src_v)
        pltpu.sync_copy(dstA_hbm.at[s], dst_v)

        @pl.loop(0, NB0)
        def _batch(b):
            pltpu.async_copy(y_hbm.at[src_v.at[b]], rows_v, sem).wait()
            pltpu.sync_copy(rows_v, accum.at[dst_v.at[b]], add=True)

    @pl.when(c == 1)
    def _sc1():
        pltpu.sync_copy(srcB_hbm.at[s], src_v.at[pl.ds(0, NB1), :])
        pltpu.sync_copy(dstB_hbm.at[s], dst_v.at[pl.ds(0, NB1), :])

        @pl.loop(0, NB1)
        def _batch(b):
            pltpu.async_copy(y_hbm.at[src_v.at[b]], rows_v, sem).wait()
            pltpu.sync_copy(rows_v, accum.at[dst_v.at[b]], add=True)

    plsc.subcore_barrier()

    @pl.loop(0, RPT // BATCH)
    def _read(k):
        pltpu.sync_copy(
            accum.at[pl.ds(s * RPT + k * BATCH, BATCH), :], rows_v)
        pltpu.sync_copy(
            rows_v, parts_out.at[c, pl.ds(s * RPT + k * BATCH, BATCH), :])


_prop = pl.kernel(
    _prop_body,
    out_type=jax.ShapeDtypeStruct((NC, N_PAD, D), jnp.float32),
    mesh=_mesh,
    compiler_params=_sc_params,
    scratch_types=[
        pltpu.VMEM((NB0, BATCH), jnp.int32),
        pltpu.VMEM((NB0, BATCH), jnp.int32),
        pltpu.VMEM((BATCH, D), jnp.float32),
        pltpu.VMEM_SHARED((N_PAD, D), jnp.float32),
        pltpu.SemaphoreType.DMA,
    ],
)


BR = 1000


def _scale_body(degT_ref, x_ref, y_ref):
    deg = jnp.sum(degT_ref[...], axis=1, keepdims=True) + 1.0
    y_ref[...] = x_ref[...] * lax.rsqrt(deg)


def _scale(degT, x):
    grid = N // BR
    return pl.pallas_call(
        _scale_body,
        grid=(grid,),
        in_specs=[
            pl.BlockSpec((BR, NW), lambda i: (i, 0)),
            pl.BlockSpec((BR, D), lambda i: (i, 0)),
        ],
        out_specs=pl.BlockSpec((BR, D), lambda i: (i, 0)),
        out_shape=jax.ShapeDtypeStruct((N, D), jnp.float32),
    )(degT, x)


def _cell_body(degT_ref, sa_ref, sb_ref, v_ref, wc0_ref, bc0_ref, wl0_ref,
               bl0_ref, wc2_ref, bc2_ref, wl2_ref, bl2_ref, *out_refs,
               relu_out, emit_y):
    deg = jnp.sum(degT_ref[...], axis=1, keepdims=True) + 1.0
    dinv = lax.rsqrt(deg)
    v = v_ref[...]
    px = dinv * (sa_ref[...] + sb_ref[...]) + v / deg
    t = jnp.dot(px, wc0_ref[...], preferred_element_type=jnp.float32)
    t = t + bc0_ref[...]
    z = jax.nn.sigmoid(
        jnp.dot(t, wl0_ref[...], preferred_element_type=jnp.float32)
        + bl0_ref[...])
    u = jnp.dot(px, wc2_ref[...], preferred_element_type=jnp.float32)
    u = u + bc2_ref[...]
    ht = jnp.tanh(
        jnp.dot(u, wl2_ref[...], preferred_element_type=jnp.float32)
        + bl2_ref[...])
    h = (1.0 - z) * ht
    if relu_out:
        h = jnp.maximum(h, 0.0)
    out_refs[0][...] = h
    if emit_y:
        out_refs[1][...] = h * dinv


def _cell(degT, sa, sb, v, Wc, bc, Wl, bl, relu_out, emit_y):
    grid = N // BR
    row = lambda i: (i, 0)
    full = lambda i: (0, 0)
    out_shape = [jax.ShapeDtypeStruct((N, D), jnp.float32)]
    out_specs = [pl.BlockSpec((BR, D), row)]
    if emit_y:
        out_shape.append(jax.ShapeDtypeStruct((N, D), jnp.float32))
        out_specs.append(pl.BlockSpec((BR, D), row))
    body = functools.partial(_cell_body, relu_out=relu_out, emit_y=emit_y)
    return pl.pallas_call(
        body,
        grid=(grid,),
        in_specs=[
            pl.BlockSpec((BR, NW), row),
            pl.BlockSpec((BR, D), row),
            pl.BlockSpec((BR, D), row),
            pl.BlockSpec((BR, D), row),
            pl.BlockSpec((D, D), full),
            pl.BlockSpec((1, D), full),
            pl.BlockSpec((D, D), full),
            pl.BlockSpec((1, D), full),
            pl.BlockSpec((D, D), full),
            pl.BlockSpec((1, D), full),
            pl.BlockSpec((D, D), full),
            pl.BlockSpec((1, D), full),
        ],
        out_specs=out_specs,
        out_shape=out_shape,
    )(degT, sa, sb, v,
      Wc[0], bc[0].reshape(1, D), Wl[0][:D], bl[0].reshape(1, D),
      Wc[2], bc[2].reshape(1, D), Wl[2][:D], bl[2].reshape(1, D))


def kernel(x, edge_index, Wc1, bc1, Wl1, bl1, Wc2, bc2, Wl2, bl2):
    src = edge_index[0]
    dst = edge_index[1]
    pad = E_PAD - E
    src3 = jnp.concatenate(
        [src, jnp.zeros((pad,), jnp.int32)]).reshape(NW, NB, BATCH)
    dst3 = jnp.concatenate(
        [dst, jnp.full((pad,), N, jnp.int32)]).reshape(NW, NB, BATCH)
    na = NS * NB0 * BATCH
    ntot = na + NS * NB1 * BATCH
    padg = ntot - E
    srcp = jnp.concatenate([src, jnp.zeros((padg,), jnp.int32)])
    dstp = jnp.concatenate([dst, jnp.full((padg,), N, jnp.int32)])
    srcA = srcp[:na].reshape(NS, NB0, BATCH)
    dstA = dstp[:na].reshape(NS, NB0, BATCH)
    srcB = srcp[na:].reshape(NS, NB1, BATCH)
    dstB = dstp[na:].reshape(NS, NB1, BATCH)

    deg_parts = _hist(dst3).reshape(NW, N_PAD)
    degT = jnp.transpose(deg_parts)[:N]

    y1 = _scale(degT, x)
    parts1 = _prop(y1, srcA, dstA, srcB, dstB)
    h1, y2 = _cell(degT, parts1[0, :N], parts1[1, :N], x, Wc1, bc1, Wl1, bl1,
                   relu_out=True, emit_y=True)
    parts2 = _prop(y2, srcA, dstA, srcB, dstB)
    (h2,) = _cell(degT, parts2[0, :N], parts2[1, :N], h1, Wc2, bc2, Wl2, bl2,
                  relu_out=False, emit_y=False)
    return h2


# speedup vs baseline: 1.0437x; 1.0403x over previous
import functools

import jax
import jax.numpy as jnp
from jax import lax
from jax.experimental import pallas as pl
from jax.experimental.pallas import tpu as pltpu
from jax.experimental.pallas import tpu_sc as plsc

N = 10000
E = 320000
D = 128
NC = 2
NS = 16
NW = NC * NS
BATCH = 128
NB = 79
EPT = NB * BATCH
NB0 = 103
NB1 = 55
E_PAD = NW * EPT
N_PAD = 10240
RPT = N_PAD // NS

_mesh = plsc.VectorSubcoreMesh(core_axis_name="c", subcore_axis_name="s")


def _hist_body(dst_hbm, deg_out, idx_v, hist_v):
    c = lax.axis_index("c")
    s = lax.axis_index("s")
    w = c * NS + s
    pltpu.sync_copy(dst_hbm.at[w], idx_v)
    zero16 = jnp.zeros((16,), jnp.float32)

    @pl.loop(0, N_PAD // 16, unroll=8)
    def _zero(i):
        hist_v[0, pl.ds(i * 16, 16)] = zero16

    ones = jnp.ones((16,), jnp.float32)
    zidx = jnp.zeros((16,), jnp.int32)

    @pl.loop(0, NB)
    def _batch(b):
        @pl.loop(0, BATCH // 16)
        def _chunk(j):
            idx = idx_v[b, pl.ds(j * 16, 16)]
            plsc.addupdate_scatter(hist_v, [zidx, idx], ones)

    pltpu.sync_copy(hist_v, deg_out.at[w])


_sc_params = pltpu.CompilerParams(needs_layout_passes=False)

_hist = pl.kernel(
    _hist_body,
    out_type=jax.ShapeDtypeStruct((NW, 1, N_PAD), jnp.float32),
    mesh=_mesh,
    compiler_params=_sc_params,
    scratch_types=[
        pltpu.VMEM((NB, BATCH), jnp.int32),
        pltpu.VMEM((1, N_PAD), jnp.float32),
    ],
)


def _prop_body(y_hbm, srcA_hbm, dstA_hbm, srcB_hbm, dstB_hbm, parts_out,
               src_v, dst_v, rows_v, accum, sem):
    c = lax.axis_index("c")
    s = lax.axis_index("s")

    zero16 = jnp.zeros((16,), jnp.float32)

    @pl.loop(0, BATCH)
    def _zrow(r):
        @pl.loop(0, D // 16, unroll=8)
        def _zcol(j):
            rows_v[r, pl.ds(j * 16, 16)] = zero16

    @pl.loop(0, RPT // BATCH)
    def _zchunk(k):
        pltpu.sync_copy(
            rows_v, accum.at[pl.ds(s * RPT + k * BATCH, BATCH), :])

    plsc.subcore_barrier()

    @pl.when(c == 0)
    def _sc0():
        pltpu.sync_copy(srcA_hbm.at[s], src_v)
        pltpu.sync_copy(dstA_hbm.at[s], dst_v)

        @pl.loop(0, NB0)
        def _batch(b):
            pltpu.async_copy(y_hbm.at[src_v.at[b]], rows_v, sem).wait()
            pltpu.sync_copy(rows_v, accum.at[dst_v.at[b]], add=True)

    @pl.when(c == 1)
    def _sc1():
        pltpu.sync_copy(srcB_hbm.at[s], src_v.at[pl.ds(0, NB1), :])
        pltpu.sync_copy(dstB_hbm.at[s], dst_v.at[pl.ds(0, NB1), :])

        @pl.loop(0, NB1)
        def _batch(b):
            pltpu.async_copy(y_hbm.at[src_v.at[b]], rows_v, sem).wait()
            pltpu.sync_copy(rows_v, accum.at[dst_v.at[b]], add=True)

    plsc.subcore_barrier()

    @pl.loop(0, RPT // BATCH)
    def _read(k):
        pltpu.sync_copy(
            accum.at[pl.ds(s * RPT + k * BATCH, BATCH), :], rows_v)
        pltpu.sync_copy(
            rows_v, parts_out.at[c, pl.ds(s * RPT + k * BATCH, BATCH), :])


_prop = pl.kernel(
    _prop_body,
    out_type=jax.ShapeDtypeStruct((NC, N_PAD, D), jnp.float32),
    mesh=_mesh,
    compiler_params=_sc_params,
    scratch_types=[
        pltpu.VMEM((NB0, BATCH), jnp.int32),
        pltpu.VMEM((NB0, BATCH), jnp.int32),
        pltpu.VMEM((BATCH, D), jnp.float32),
        pltpu.VMEM_SHARED((N_PAD, D), jnp.float32),
        pltpu.SemaphoreType.DMA,
    ],
)


BR = 1000


def _scale_body(degT_ref, x_ref, y_ref):
    deg = jnp.sum(degT_ref[...], axis=1, keepdims=True) + 1.0
    y_ref[...] = x_ref[...] * lax.rsqrt(deg)


def _scale(degT, x):
    grid = N // BR
    return pl.pallas_call(
        _scale_body,
        grid=(grid,),
        in_specs=[
            pl.BlockSpec((BR, NW), lambda i: (i, 0)),
            pl.BlockSpec((BR, D), lambda i: (i, 0)),
        ],
        out_specs=pl.BlockSpec((BR, D), lambda i: (i, 0)),
        out_shape=jax.ShapeDtypeStruct((N, D), jnp.float32),
    )(degT, x)


def _cell_body(degT_ref, parts_ref, v_ref, wc0_ref, bc0_ref, wl0_ref,
               bl0_ref, wc2_ref, bc2_ref, wl2_ref, bl2_ref, *out_refs,
               relu_out, emit_y):
    deg = jnp.sum(degT_ref[...], axis=1, keepdims=True) + 1.0
    dinv = lax.rsqrt(deg)
    v = v_ref[...]
    px = dinv * (parts_ref[0] + parts_ref[1]) + v / deg
    t = jnp.dot(px, wc0_ref[...], preferred_element_type=jnp.float32)
    t = t + bc0_ref[...]
    z = jax.nn.sigmoid(
        jnp.dot(t, wl0_ref[...], preferred_element_type=jnp.float32)
        + bl0_ref[...])
    u = jnp.dot(px, wc2_ref[...], preferred_element_type=jnp.float32)
    u = u + bc2_ref[...]
    ht = jnp.tanh(
        jnp.dot(u, wl2_ref[...], preferred_element_type=jnp.float32)
        + bl2_ref[...])
    h = (1.0 - z) * ht
    if relu_out:
        h = jnp.maximum(h, 0.0)
    out_refs[0][...] = h
    if emit_y:
        out_refs[1][...] = h * dinv


def _cell(degT, parts, v, Wc, bc, Wl, bl, relu_out, emit_y):
    grid = N // BR
    row = lambda i: (i, 0)
    row3 = lambda i: (0, i, 0)
    full = lambda i: (0, 0)
    out_shape = [jax.ShapeDtypeStruct((N, D), jnp.float32)]
    out_specs = [pl.BlockSpec((BR, D), row)]
    if emit_y:
        out_shape.append(jax.ShapeDtypeStruct((N, D), jnp.float32))
        out_specs.append(pl.BlockSpec((BR, D), row))
    body = functools.partial(_cell_body, relu_out=relu_out, emit_y=emit_y)
    return pl.pallas_call(
        body,
        grid=(grid,),
        in_specs=[
            pl.BlockSpec((BR, NW), row),
            pl.BlockSpec((NC, BR, D), row3),
            pl.BlockSpec((BR, D), row),
            pl.BlockSpec((D, D), full),
            pl.BlockSpec((1, D), full),
            pl.BlockSpec((D, D), full),
            pl.BlockSpec((1, D), full),
            pl.BlockSpec((D, D), full),
            pl.BlockSpec((1, D), full),
            pl.BlockSpec((D, D), full),
            pl.BlockSpec((1, D), full),
        ],
        out_specs=out_specs,
        out_shape=out_shape,
    )(degT, parts, v,
      Wc[0], bc[0].reshape(1, D), Wl[0][:D], bl[0].reshape(1, D),
      Wc[2], bc[2].reshape(1, D), Wl[2][:D], bl[2].reshape(1, D))


def kernel(x, edge_index, Wc1, bc1, Wl1, bl1, Wc2, bc2, Wl2, bl2):
    src = edge_index[0]
    dst = edge_index[1]
    pad = E_PAD - E
    src3 = jnp.concatenate(
        [src, jnp.zeros((pad,), jnp.int32)]).reshape(NW, NB, BATCH)
    dst3 = jnp.concatenate(
        [dst, jnp.full((pad,), N, jnp.int32)]).reshape(NW, NB, BATCH)
    na = NS * NB0 * BATCH
    ntot = na + NS * NB1 * BATCH
    padg = ntot - E
    srcp = jnp.concatenate([src, jnp.zeros((padg,), jnp.int32)])
    dstp = jnp.concatenate([dst, jnp.full((padg,), N, jnp.int32)])
    srcA = srcp[:na].reshape(NS, NB0, BATCH)
    dstA = dstp[:na].reshape(NS, NB0, BATCH)
    srcB = srcp[na:].reshape(NS, NB1, BATCH)
    dstB = dstp[na:].reshape(NS, NB1, BATCH)

    deg_parts = _hist(dst3).reshape(NW, N_PAD)
    degT = jnp.transpose(deg_parts)[:N]

    y1 = _scale(degT, x)
    parts1 = _prop(y1, srcA, dstA, srcB, dstB)
    h1, y2 = _cell(degT, parts1, x, Wc1, bc1, Wl1, bl1,
                   relu_out=True, emit_y=True)
    parts2 = _prop(y2, srcA, dstA, srcB, dstB)
    (h2,) = _cell(degT, parts2, h1, Wc2, bc2, Wl2, bl2,
                  relu_out=False, emit_y=False)
    return h2
